# baseline (speedup 1.0000x reference)
import jax
import jax.numpy as jnp
from jax import lax
from jax.experimental import pallas as pl
from jax.experimental.pallas import tpu as pltpu

_EPS = 0.01
_N = 4096
_BLK = 256
_G = _BLK // 64


def _fill(buf, i, x_ref):
    buf[...] = jnp.zeros((_BLK, 1, 64, 64), jnp.float32)
    slab = jnp.where(
        lax.broadcasted_iota(jnp.int32, (64, 1, 1, 64), 0)
        == lax.broadcasted_iota(jnp.int32, (64, 1, 1, 64), 3),
        _EPS, 0.0).astype(jnp.float32)
    for g in range(_G - 1):
        buf[pl.ds(64 * g + 1, 64), :, pl.ds(_G * i + g, 1), :] = slab
    buf[pl.ds(64 * (_G - 1) + 1, 63), :, pl.ds(_G * i + _G - 1, 1), :] = (
        slab[:63])

    @pl.when(i > 0)
    def _():
        buf[pl.ds(0, 1), :, pl.ds(_G * i - 1, 1), :] = jnp.where(
            lax.broadcasted_iota(jnp.int32, (1, 1, 1, 64), 3) == 63,
            _EPS, 0.0).astype(jnp.float32)

    @pl.when(i == 0)
    def _():
        buf[pl.ds(0, 1), :, :, :] = x_ref[...]


def _toz_body(x_ref, o_ref, s0, s1, st, sem0, sem1, semt):
    i = pl.program_id(0)
    bufs = (s0, s1)
    sems = (sem0, sem1)
    for p in range(2):
        @pl.when(i % 2 == p)
        def _(p=p):
            buf, sem = bufs[p], sems[p]

            @pl.when(i >= 2)
            def _():
                pltpu.make_async_copy(
                    buf, o_ref.at[pl.ds(i * _BLK, _BLK)], sem).wait()

            _fill(buf, i, x_ref)
            pltpu.make_async_copy(
                buf, o_ref.at[pl.ds(i * _BLK, _BLK)], sem).start()

    @pl.when(i == 15)
    def _():
        st[...] = jnp.where(
            (lax.broadcasted_iota(jnp.int32, (1, 1, 64, 64), 2) == 63)
            & (lax.broadcasted_iota(jnp.int32, (1, 1, 64, 64), 3) == 63),
            _EPS, 0.0).astype(jnp.float32)
        pltpu.make_async_copy(st, o_ref.at[pl.ds(_N, 1)], semt).start()
        pltpu.make_async_copy(
            s0, o_ref.at[pl.ds(14 * _BLK, _BLK)], sem0).wait()
        pltpu.make_async_copy(
            s1, o_ref.at[pl.ds(15 * _BLK, _BLK)], sem1).wait()
        pltpu.make_async_copy(st, o_ref.at[pl.ds(_N, 1)], semt).wait()


def kernel(x):
    out = pl.pallas_call(
        _toz_body,
        grid=(16,),
        in_specs=[pl.BlockSpec((1, 1, 64, 64), lambda i: (0, 0, 0, 0))],
        out_specs=pl.BlockSpec(memory_space=pl.ANY),
        out_shape=jax.ShapeDtypeStruct((_N + 1, 1, 64, 64), jnp.float32),
        scratch_shapes=[
            pltpu.VMEM((_BLK, 1, 64, 64), jnp.float32),
            pltpu.VMEM((_BLK, 1, 64, 64), jnp.float32),
            pltpu.VMEM((1, 1, 64, 64), jnp.float32),
            pltpu.SemaphoreType.DMA,
            pltpu.SemaphoreType.DMA,
            pltpu.SemaphoreType.DMA,
        ],
    )(x)
    return out
